# baseline (speedup 1.0000x reference)
import jax
import jax.numpy as jnp
from jax import lax
from jax.experimental import pallas as pl
from jax.experimental.pallas import tpu as pltpu
from jax.experimental.pallas import tpu_sc as plsc

_N = 10000
_E = 160000
_D = 256
_DH = _D // 2
_K = 128
_NS = 16
_NC = 2
_CPT = 80
_EPAD = _CPT * _NS * _K
_RPT = 624
_NB = 3
_NM = 4


def _sc_spmm_body(xr_hbm, g0_hbm, g1_hbm, row_hbm, val_hbm, out_hbm,
                  acc, buf0, buf1, colr0, colr1, rowr0, rowr1,
                  valr0, valr1, compact, rowc, rowtmp,
                  gsem0, gsem1, msem0, msem1):
    c = lax.axis_index("c")
    s = lax.axis_index("s")
    base = s * _CPT
    bufs = (buf0, buf1)
    colr = (colr0, colr1)
    rowr = (rowr0, rowr1)
    valr = (valr0, valr1)
    gsems = (gsem0, gsem1)
    msems = (msem0, msem1)
    zero16 = jnp.zeros((16,), jnp.float32)

    @pl.loop(0, _K)
    def _zero(e):
        for d in range(_DH // 16):
            buf0[e, pl.ds(d * 16, 16)] = zero16
            compact[e, pl.ds(d * 16, 16)] = zero16

    for r in range(8):
        rowc[r, pl.ds(0, 16)] = jnp.zeros((16,), jnp.int32)

    @pl.loop(0, 4)
    def _zinit(i):
        pltpu.sync_copy(buf0, acc.at[pl.ds(s * _RPT + i * _K, _K)])

    @pl.when(s < 15)
    def _():
        pltpu.sync_copy(buf0.at[pl.ds(0, 112)],
                        acc.at[pl.ds(s * _RPT + 4 * _K, 112)])

    @pl.when(s == 15)
    def _():
        pltpu.sync_copy(buf0, acc.at[pl.ds(15 * _RPT + 4 * _K, _K)])

    plsc.subcore_barrier()

    def _load_meta_sync(ci, m):
        eo = (base + ci) * _K

        @pl.when(c == 0)
        def _():
            pltpu.sync_copy(g0_hbm.at[pl.ds(eo, _K)], colr[m])

        @pl.when(c == 1)
        def _():
            pltpu.sync_copy(g1_hbm.at[pl.ds(eo, _K)], colr[m])

        pltpu.sync_copy(row_hbm.at[pl.ds(eo, _K)], rowr[m].at[0])
        pltpu.sync_copy(val_hbm.at[pl.ds(eo, _K)], valr[m])

    def _issue_meta(ci, m):
        eo = (base + ci) * _K

        @pl.when(c == 0)
        def _():
            pltpu.async_copy(g0_hbm.at[pl.ds(eo, _K)], colr[m], msems[m])

        @pl.when(c == 1)
        def _():
            pltpu.async_copy(g1_hbm.at[pl.ds(eo, _K)], colr[m], msems[m])

        pltpu.async_copy(row_hbm.at[pl.ds(eo, _K)], rowr[m].at[0], msems[m])
        pltpu.async_copy(val_hbm.at[pl.ds(eo, _K)], valr[m], msems[m])

    def _wait_meta(ci, m):
        eo = (base + ci) * _K
        pltpu.make_async_copy(g0_hbm.at[pl.ds(eo, _K)], colr[m],
                              msems[m]).wait()
        pltpu.make_async_copy(row_hbm.at[pl.ds(eo, _K)], rowr[m].at[0],
                              msems[m]).wait()
        pltpu.make_async_copy(val_hbm.at[pl.ds(eo, _K)], valr[m],
                              msems[m]).wait()

    def _issue_gather(m, b):
        pltpu.async_copy(xr_hbm.at[colr[m]], bufs[b], gsems[b])

    def _step(ci, b):
        buf = bufs[b]
        m = b
        pltpu.make_async_copy(xr_hbm.at[colr[m]], buf, gsems[b]).wait()

        @pl.when(ci + 1 < _CPT)
        def _():
            _wait_meta(ci + 1, 1 - m)
            _issue_gather(1 - m, 1 - b)

        iota16 = lax.iota(jnp.int32, 16)

        @pl.loop(0, _K // 16, init_carry=(jnp.int32(-1), jnp.int32(0)))
        def _combine(g, carry):
            carry_row, run_base = carry
            row16 = rowr[m][0, pl.ds(g * 16, 16)]
            val16 = valr[m][pl.ds(g * 16, 16)]
            rowtmp[pl.ds(0, 16)] = jnp.full((16,), carry_row, jnp.int32)
            rowtmp[pl.ds(1, 16)] = row16
            prev16 = rowtmp[pl.ds(0, 16)]
            runinc = jnp.where(row16 != prev16, 1, 0).astype(jnp.int32)
            for k in (1, 2, 4, 8):
                rowtmp[pl.ds(0, 16)] = jnp.zeros((16,), jnp.int32)
                rowtmp[pl.ds(k, 16)] = runinc
                runinc = runinc + rowtmp[pl.ds(0, 16)]
            run16 = run_base + runinc - 1
            r0 = lax.shift_right_logical(run16[0], 4)
            r1 = lax.shift_right_logical(run16[15], 4)
            for r in (r0, r1):
                lane = run16 - r * 16
                v = rowc[r, pl.ds(0, 16)]
                for j in range(16):
                    v = jnp.where(iota16 == lane[j],
                                  jnp.full((16,), row16[j], jnp.int32), v)
                rowc[r, pl.ds(0, 16)] = v
            for j in range(16):
                e = g * 16 + j
                rid = run16[j]
                vb = jnp.full((16,), val16[j], jnp.float32)
                for d in range(_DH // 16):
                    sl = pl.ds(d * 16, 16)
                    compact[rid, sl] = compact[rid, sl] + buf[e, sl] * vb
            return row16[15], run16[15] + 1

        nr = _combine[1]

        @pl.when(ci + 2 < _CPT)
        def _():
            _issue_meta(ci + 2, m)

        for blk in range(8):
            @pl.when(blk * 16 < nr)
            def _():
                pltpu.sync_copy(compact.at[pl.ds(blk * 16, 16)],
                                acc.at[rowc.at[blk]], add=True)

        @pl.loop(0, nr)
        def _rezero(r):
            for d in range(_DH // 16):
                compact[r, pl.ds(d * 16, 16)] = zero16

    _load_meta_sync(0, 0)
    _issue_meta(1, 1)
    _issue_gather(0, 0)

    @pl.loop(0, _CPT, step=2)
    def _chunk(i):
        _step(i, 0)
        _step(i + 1, 1)

    plsc.subcore_barrier()

    @pl.when(s < 15)
    def _():
        pltpu.sync_copy(acc.at[pl.ds(s * _RPT, _RPT)],
                        out_hbm.at[c, pl.ds(s * _RPT, _RPT)])

    @pl.when(s == 15)
    def _():
        pltpu.sync_copy(acc.at[pl.ds(15 * _RPT, 640)],
                        out_hbm.at[c, pl.ds(15 * _RPT, 640)])


@jax.jit
def _sc_spmm(xr, g0, g1, row1d, val1d):
    mesh = plsc.VectorSubcoreMesh(core_axis_name="c", subcore_axis_name="s")
    fn = pl.kernel(
        _sc_spmm_body,
        out_type=jax.ShapeDtypeStruct((_NC, _N, _DH), jnp.float32),
        mesh=mesh,
        scratch_types=(
            [pltpu.VMEM_SHARED((_N, _DH), jnp.float32)]
            + [pltpu.VMEM((_K, _DH), jnp.float32)] * 2
            + [pltpu.VMEM((_K,), jnp.int32)] * 2
            + [pltpu.VMEM((1, _K), jnp.int32)] * 2
            + [pltpu.VMEM((_K,), jnp.float32)] * 2
            + [pltpu.VMEM((_K, _DH), jnp.float32)]
            + [pltpu.VMEM((8, 16), jnp.int32)]
            + [pltpu.VMEM((32,), jnp.int32)]
            + [pltpu.SemaphoreType.DMA] * 4
        ),
    )
    return fn(xr, g0, g1, row1d, val1d)


def _tc_norm_body(sum_ref, o_ref):
    a = sum_ref[0]
    b = sum_ref[1]
    sq = (jnp.sum(a * a, axis=1) + jnp.sum(b * b, axis=1)
          - 2.0 * a[:, 0] * a[:, 0])
    coeff = 1.0 / jnp.sqrt(jnp.abs(sq))
    o_ref[:, : _DH] = a * coeff[:, None]
    o_ref[:, _DH:] = b * coeff[:, None]


@jax.jit
def _tc_norm(sums):
    blk = 2000
    return pl.pallas_call(
        _tc_norm_body,
        grid=(_N // blk,),
        in_specs=[pl.BlockSpec((_NC, blk, _DH), lambda i: (0, i, 0))],
        out_specs=pl.BlockSpec((blk, _D), lambda i: (i, 0)),
        out_shape=jax.ShapeDtypeStruct((_N, _D), jnp.float32),
    )(sums)


def kernel(x, adj_indices, adj_values):
    perm = jnp.argsort(adj_indices[0])
    row = adj_indices[0][perm]
    col = adj_indices[1][perm]
    adj_values = adj_values[perm]
    pad = _EPAD - _E
    row1d = jnp.pad(row, (0, pad))
    val1d = jnp.pad(adj_values, (0, pad))
    g0 = jnp.pad(col * 2, (0, pad))
    g1 = jnp.pad(col * 2 + 1, (0, pad))
    xr = x.reshape(2 * _N, _DH)
    sums = _sc_spmm(xr, g0, g1, row1d, val1d)
    return _tc_norm(sums)


# speedup vs baseline: 2.1366x; 2.1366x over previous
import jax
import jax.numpy as jnp
from jax import lax
from jax.experimental import pallas as pl
from jax.experimental.pallas import tpu as pltpu
from jax.experimental.pallas import tpu_sc as plsc

_N = 10000
_E = 160000
_D = 256
_DH = _D // 2
_K = 128
_NS = 16
_NC = 2
_CPT = 80
_EPAD = _CPT * _NS * _K
_RPT = 624
_NB = 3
_NM = 4


def _sc_spmm_body(xr_hbm, g0_hbm, g1_hbm, row_hbm, val_hbm, out_hbm,
                  acc, buf0, buf1,
                  colr0, colr1, colr2, colr3,
                  rowr0, rowr1, rowr2, rowr3,
                  valr0, valr1, valr2, valr3,
                  gsem0, gsem1, msem0, msem1, msem2, msem3):
    c = lax.axis_index("c")
    s = lax.axis_index("s")
    base = s * _CPT
    bufs = (buf0, buf1)
    colr = (colr0, colr1, colr2, colr3)
    rowr = (rowr0, rowr1, rowr2, rowr3)
    valr = (valr0, valr1, valr2, valr3)
    gsems = (gsem0, gsem1)
    msems = (msem0, msem1, msem2, msem3)

    @pl.loop(0, _K)
    def _zero(e):
        for d in range(_DH // 16):
            buf0[e, pl.ds(d * 16, 16)] = jnp.zeros((16,), jnp.float32)

    @pl.loop(0, 4)
    def _zinit(i):
        pltpu.sync_copy(buf0, acc.at[pl.ds(s * _RPT + i * _K, _K)])

    @pl.when(s < 15)
    def _():
        pltpu.sync_copy(buf0.at[pl.ds(0, 112)],
                        acc.at[pl.ds(s * _RPT + 4 * _K, 112)])

    @pl.when(s == 15)
    def _():
        pltpu.sync_copy(buf0, acc.at[pl.ds(15 * _RPT + 4 * _K, _K)])

    plsc.subcore_barrier()

    def _load_meta_sync(ci, m):
        eo = (base + ci) * _K

        @pl.when(c == 0)
        def _():
            pltpu.sync_copy(g0_hbm.at[pl.ds(eo, _K)], colr[m])

        @pl.when(c == 1)
        def _():
            pltpu.sync_copy(g1_hbm.at[pl.ds(eo, _K)], colr[m])

        pltpu.sync_copy(row_hbm.at[pl.ds(eo, _K)], rowr[m].at[0])
        pltpu.sync_copy(val_hbm.at[pl.ds(eo, _K)], valr[m])

    def _issue_meta(ci, m):
        eo = (base + ci) * _K

        @pl.when(c == 0)
        def _():
            pltpu.async_copy(g0_hbm.at[pl.ds(eo, _K)], colr[m], msems[m])

        @pl.when(c == 1)
        def _():
            pltpu.async_copy(g1_hbm.at[pl.ds(eo, _K)], colr[m], msems[m])

        pltpu.async_copy(row_hbm.at[pl.ds(eo, _K)], rowr[m].at[0], msems[m])
        pltpu.async_copy(val_hbm.at[pl.ds(eo, _K)], valr[m], msems[m])

    def _wait_meta(ci, m):
        eo = (base + ci) * _K
        pltpu.make_async_copy(g0_hbm.at[pl.ds(eo, _K)], colr[m],
                              msems[m]).wait()
        pltpu.make_async_copy(row_hbm.at[pl.ds(eo, _K)], rowr[m].at[0],
                              msems[m]).wait()
        pltpu.make_async_copy(val_hbm.at[pl.ds(eo, _K)], valr[m],
                              msems[m]).wait()

    def _issue_gather(m, b):
        pltpu.async_copy(xr_hbm.at[colr[m]], bufs[b], gsems[b])

    def _step(ci, b, m):
        buf = bufs[b]
        pltpu.make_async_copy(xr_hbm.at[colr[m]], buf, gsems[b]).wait()

        @pl.when(ci + 1 < _CPT)
        def _():
            _wait_meta(ci + 1, (m + 1) % _NM)
            _issue_gather((m + 1) % _NM, 1 - b)

        @pl.when(ci + 3 < _CPT)
        def _():
            _issue_meta(ci + 3, (m + 3) % _NM)

        @pl.loop(0, _K // 16)
        def _scale(g):
            val16 = valr[m][pl.ds(g * 16, 16)]
            for j in range(16):
                e = g * 16 + j
                vb = jnp.full((16,), val16[j], jnp.float32)
                for d in range(_DH // 16):
                    sl = pl.ds(d * 16, 16)
                    buf[e, sl] = buf[e, sl] * vb

        pltpu.sync_copy(buf, acc.at[rowr[m].at[0]], add=True)

    _load_meta_sync(0, 0)
    _issue_meta(1, 1)
    _issue_meta(2, 2)
    _issue_gather(0, 0)

    @pl.loop(0, _CPT, step=4)
    def _chunk(i):
        for u in range(4):
            _step(i + u, u % 2, u % _NM)

    plsc.subcore_barrier()

    @pl.when(s < 15)
    def _():
        pltpu.sync_copy(acc.at[pl.ds(s * _RPT, _RPT)],
                        out_hbm.at[c, pl.ds(s * _RPT, _RPT)])

    @pl.when(s == 15)
    def _():
        pltpu.sync_copy(acc.at[pl.ds(15 * _RPT, 640)],
                        out_hbm.at[c, pl.ds(15 * _RPT, 640)])


@jax.jit
def _sc_spmm(xr, g0, g1, row1d, val1d):
    mesh = plsc.VectorSubcoreMesh(core_axis_name="c", subcore_axis_name="s")
    fn = pl.kernel(
        _sc_spmm_body,
        out_type=jax.ShapeDtypeStruct((_NC, _N, _DH), jnp.float32),
        mesh=mesh,
        scratch_types=(
            [pltpu.VMEM_SHARED((_N, _DH), jnp.float32)]
            + [pltpu.VMEM((_K, _DH), jnp.float32)] * 2
            + [pltpu.VMEM((_K,), jnp.int32)] * 4
            + [pltpu.VMEM((1, _K), jnp.int32)] * 4
            + [pltpu.VMEM((_K,), jnp.float32)] * 4
            + [pltpu.SemaphoreType.DMA] * 6
        ),
    )
    return fn(xr, g0, g1, row1d, val1d)


def _tc_norm_body(sum_ref, o_ref):
    a = sum_ref[0]
    b = sum_ref[1]
    sq = (jnp.sum(a * a, axis=1) + jnp.sum(b * b, axis=1)
          - 2.0 * a[:, 0] * a[:, 0])
    coeff = 1.0 / jnp.sqrt(jnp.abs(sq))
    o_ref[:, : _DH] = a * coeff[:, None]
    o_ref[:, _DH:] = b * coeff[:, None]


@jax.jit
def _tc_norm(sums):
    blk = 2000
    return pl.pallas_call(
        _tc_norm_body,
        grid=(_N // blk,),
        in_specs=[pl.BlockSpec((_NC, blk, _DH), lambda i: (0, i, 0))],
        out_specs=pl.BlockSpec((blk, _D), lambda i: (i, 0)),
        out_shape=jax.ShapeDtypeStruct((_N, _D), jnp.float32),
    )(sums)


def kernel(x, adj_indices, adj_values):
    row = adj_indices[0]
    col = adj_indices[1]
    pad = _EPAD - _E
    row1d = jnp.pad(row, (0, pad))
    val1d = jnp.pad(adj_values, (0, pad))
    g0 = jnp.pad(col * 2, (0, pad))
    g1 = jnp.pad(col * 2 + 1, (0, pad))
    xr = x.reshape(2 * _N, _DH)
    sums = _sc_spmm(xr, g0, g1, row1d, val1d)
    return _tc_norm(sums)
